# baseline (speedup 1.0000x reference)
import functools

import jax
import jax.numpy as jnp
from jax import lax
from jax.experimental import pallas as pl
from jax.experimental.pallas import tpu as pltpu
from jax.experimental.pallas import tpu_sc as plsc

_B = 8
_R = 512
_C = 512
_RT = 352

_NC = 2
_NS = 16
_L = 16
_NW = _NC * _NS
_WPB = _NW // _B
_ROWS_W = (_R - _RT) // _WPB
_U = 2
_VPW = _ROWS_W * _C // _L


def _sc_body(out_hbm, tgt_hbm, res_hbm, obuf, tbuf, res_v, sem):
    wid = lax.axis_index("s") * _NC + lax.axis_index("c")
    b = wid // _WPB
    r0 = _RT + (wid % _WPB) * _ROWS_W
    rows = pl.ds(r0, _ROWS_W)
    h0 = pltpu.async_copy(out_hbm.at[b, rows, :], obuf, sem)
    h1 = pltpu.async_copy(tgt_hbm.at[b, rows, :], tbuf, sem)
    h0.wait()
    h1.wait()

    def body(i, accs):
        s0, s1, c0, c1 = accs
        base = i * (_L * _U)
        for u in range(_U):
            off = base + u * _L
            r = off // _C
            col = off % _C
            tv = tbuf[r, pl.ds(col, _L)]
            ov = obuf[r, pl.ds(col, _L)]
            m = tv > 0.0
            e = jnp.where(m, jnp.abs(tv - ov), 0.0)
            pc = jnp.where(m, 1.0, 0.0)
            if u % 2 == 0:
                s0 = s0 + e
                c0 = c0 + pc
            else:
                s1 = s1 + e
                c1 = c1 + pc
        return s0, s1, c0, c1

    z = jnp.zeros((_L,), jnp.float32)
    s0, s1, c0, c1 = lax.fori_loop(0, _VPW // _U, body, (z, z, z, z))
    res_v[0, :] = s0 + s1
    res_v[1, :] = c0 + c1
    pltpu.sync_copy(res_v, res_hbm.at[wid])


@functools.cache
def _make_sc():
    mesh = plsc.VectorSubcoreMesh(core_axis_name="c", subcore_axis_name="s")
    return pl.kernel(
        _sc_body,
        out_type=jax.ShapeDtypeStruct((_NW, 2, _L), jnp.float32),
        mesh=mesh,
        compiler_params=pltpu.CompilerParams(use_tc_tiling_on_sc=True),
        scratch_types=[
            pltpu.VMEM((_ROWS_W, _C), jnp.float32),
            pltpu.VMEM((_ROWS_W, _C), jnp.float32),
            pltpu.VMEM((2, _L), jnp.float32),
            pltpu.SemaphoreType.DMA,
        ],
    )


def _tc_body(o_ref, t_ref, s_ref, c_ref):
    b = pl.program_id(0)
    t = t_ref[0]
    o = o_ref[0]
    m = t > 0.0
    e = jnp.where(m, jnp.abs(t - o), 0.0)
    mf = jnp.where(m, 1.0, 0.0)

    @pl.when(b == 0)
    def _init():
        s_ref[0, 0] = 0.0
        c_ref[0, 0] = 0.0

    s_ref[0, 0] += jnp.sum(e)
    c_ref[0, 0] += jnp.sum(mf)


@functools.cache
def _make_tc():
    return pl.pallas_call(
        _tc_body,
        grid=(_B,),
        in_specs=[
            pl.BlockSpec((1, _RT, _C), lambda b: (b, 0, 0)),
            pl.BlockSpec((1, _RT, _C), lambda b: (b, 0, 0)),
        ],
        out_specs=[
            pl.BlockSpec(memory_space=pltpu.SMEM),
            pl.BlockSpec(memory_space=pltpu.SMEM),
        ],
        out_shape=[
            jax.ShapeDtypeStruct((1, 1), jnp.float32),
            jax.ShapeDtypeStruct((1, 1), jnp.float32),
        ],
    )


def kernel(outputs, target):
    p = _make_sc()(outputs, target)
    ts, tc = _make_tc()(outputs, target)
    num = ts[0, 0] + jnp.sum(p[:, 0, :])
    den = tc[0, 0] + jnp.sum(p[:, 1, :])
    return num / den


# speedup vs baseline: 1.0255x; 1.0255x over previous
import functools

import jax
import jax.numpy as jnp
from jax import lax
from jax.experimental import pallas as pl
from jax.experimental.pallas import tpu as pltpu
from jax.experimental.pallas import tpu_sc as plsc

_B = 8
_R = 512
_C = 512
_RT = 384

_NC = 2
_NS = 16
_L = 16
_NW = _NC * _NS
_WPB = _NW // _B
_ROWS_W = (_R - _RT) // _WPB
_U = 2
_VPW = _ROWS_W * _C // _L


def _sc_body(out_hbm, tgt_hbm, res_hbm, obuf, tbuf, res_v, sem):
    wid = lax.axis_index("s") * _NC + lax.axis_index("c")
    b = wid // _WPB
    r0 = _RT + (wid % _WPB) * _ROWS_W
    rows = pl.ds(r0, _ROWS_W)
    h0 = pltpu.async_copy(out_hbm.at[b, rows, :], obuf, sem)
    h1 = pltpu.async_copy(tgt_hbm.at[b, rows, :], tbuf, sem)
    h0.wait()
    h1.wait()

    def body(i, accs):
        s0, s1, c0, c1 = accs
        base = i * (_L * _U)
        for u in range(_U):
            off = base + u * _L
            r = off // _C
            col = off % _C
            tv = tbuf[r, pl.ds(col, _L)]
            ov = obuf[r, pl.ds(col, _L)]
            m = tv > 0.0
            e = jnp.where(m, jnp.abs(tv - ov), 0.0)
            pc = jnp.where(m, 1.0, 0.0)
            if u % 2 == 0:
                s0 = s0 + e
                c0 = c0 + pc
            else:
                s1 = s1 + e
                c1 = c1 + pc
        return s0, s1, c0, c1

    z = jnp.zeros((_L,), jnp.float32)
    s0, s1, c0, c1 = lax.fori_loop(0, _VPW // _U, body, (z, z, z, z))
    res_v[0, :] = s0 + s1
    res_v[1, :] = c0 + c1
    pltpu.sync_copy(res_v, res_hbm.at[wid])


@functools.cache
def _make_sc():
    mesh = plsc.VectorSubcoreMesh(core_axis_name="c", subcore_axis_name="s")
    return pl.kernel(
        _sc_body,
        out_type=jax.ShapeDtypeStruct((_NW, 2, _L), jnp.float32),
        mesh=mesh,
        compiler_params=pltpu.CompilerParams(use_tc_tiling_on_sc=True),
        scratch_types=[
            pltpu.VMEM((_ROWS_W, _C), jnp.float32),
            pltpu.VMEM((_ROWS_W, _C), jnp.float32),
            pltpu.VMEM((2, _L), jnp.float32),
            pltpu.SemaphoreType.DMA,
        ],
    )


def _tc_body(o_ref, t_ref, s_ref, c_ref):
    b = pl.program_id(0)
    t = t_ref[0]
    o = o_ref[0]
    m = t > 0.0
    e = jnp.where(m, jnp.abs(t - o), 0.0)
    mf = jnp.where(m, 1.0, 0.0)

    @pl.when(b == 0)
    def _init():
        s_ref[0, 0] = 0.0
        c_ref[0, 0] = 0.0

    s_ref[0, 0] += jnp.sum(e)
    c_ref[0, 0] += jnp.sum(mf)


@functools.cache
def _make_tc():
    return pl.pallas_call(
        _tc_body,
        grid=(_B,),
        in_specs=[
            pl.BlockSpec((1, _RT, _C), lambda b: (b, 0, 0)),
            pl.BlockSpec((1, _RT, _C), lambda b: (b, 0, 0)),
        ],
        out_specs=[
            pl.BlockSpec(memory_space=pltpu.SMEM),
            pl.BlockSpec(memory_space=pltpu.SMEM),
        ],
        out_shape=[
            jax.ShapeDtypeStruct((1, 1), jnp.float32),
            jax.ShapeDtypeStruct((1, 1), jnp.float32),
        ],
    )


def kernel(outputs, target):
    p = _make_sc()(outputs, target)
    ts, tc = _make_tc()(outputs, target)
    num = ts[0, 0] + jnp.sum(p[:, 0, :])
    den = tc[0, 0] + jnp.sum(p[:, 1, :])
    return num / den


# speedup vs baseline: 1.0277x; 1.0021x over previous
import functools

import jax
import jax.numpy as jnp
from jax import lax
from jax.experimental import pallas as pl
from jax.experimental.pallas import tpu as pltpu
from jax.experimental.pallas import tpu_sc as plsc

_B = 8
_R = 512
_C = 512
_RT = 384

_NC = 2
_NS = 16
_L = 16
_NW = _NC * _NS
_WPB = _NW // _B
_ROWS_W = (_R - _RT) // _WPB
_U = 2
_VPW = _ROWS_W * _C // _L


def _sc_body(out_hbm, tgt_hbm, res_hbm, obuf, tbuf, res_v, sem):
    wid = lax.axis_index("s") * _NC + lax.axis_index("c")
    b = wid // _WPB
    r0 = _RT + (wid % _WPB) * _ROWS_W
    rows = pl.ds(r0, _ROWS_W)
    h0 = pltpu.async_copy(out_hbm.at[b, rows, :], obuf, sem)
    h1 = pltpu.async_copy(tgt_hbm.at[b, rows, :], tbuf, sem)
    h0.wait()
    h1.wait()

    def body(i, accs):
        s0, s1, c0, c1 = accs
        base = i * (_L * _U)
        for u in range(_U):
            off = base + u * _L
            r = off // _C
            col = off % _C
            tv = tbuf[r, pl.ds(col, _L)]
            ov = obuf[r, pl.ds(col, _L)]
            m = tv > 0.0
            e = jnp.where(m, jnp.abs(tv - ov), 0.0)
            pc = jnp.where(m, 1.0, 0.0)
            if u % 2 == 0:
                s0 = s0 + e
                c0 = c0 + pc
            else:
                s1 = s1 + e
                c1 = c1 + pc
        return s0, s1, c0, c1

    z = jnp.zeros((_L,), jnp.float32)
    s0, s1, c0, c1 = lax.fori_loop(0, _VPW // _U, body, (z, z, z, z))
    res_v[0, :] = s0 + s1
    res_v[1, :] = c0 + c1
    pltpu.sync_copy(res_v, res_hbm.at[wid])


@functools.cache
def _make_sc():
    mesh = plsc.VectorSubcoreMesh(core_axis_name="c", subcore_axis_name="s")
    return pl.kernel(
        _sc_body,
        out_type=jax.ShapeDtypeStruct((_NW, 2, _L), jnp.float32),
        mesh=mesh,
        compiler_params=pltpu.CompilerParams(use_tc_tiling_on_sc=True),
        scratch_types=[
            pltpu.VMEM((_ROWS_W, _C), jnp.float32),
            pltpu.VMEM((_ROWS_W, _C), jnp.float32),
            pltpu.VMEM((2, _L), jnp.float32),
            pltpu.SemaphoreType.DMA,
        ],
    )


def _tc_body(o_ref, t_ref, s_ref, c_ref):
    b = pl.program_id(0)
    t = t_ref[0]
    o = o_ref[0]
    m = t > 0.0
    e = jnp.where(m, jnp.abs(t - o), 0.0)
    mf = jnp.where(m, 1.0, 0.0)

    @pl.when(b == 0)
    def _init():
        s_ref[0, 0] = 0.0
        c_ref[0, 0] = 0.0

    s_ref[0, 0] += jnp.sum(e)
    c_ref[0, 0] += jnp.sum(mf)


@functools.cache
def _make_tc():
    return pl.pallas_call(
        _tc_body,
        grid=(_B,),
        in_specs=[
            pl.BlockSpec((1, _RT, _C), lambda b: (b, 0, 0)),
            pl.BlockSpec((1, _RT, _C), lambda b: (b, 0, 0)),
        ],
        out_specs=[
            pl.BlockSpec(memory_space=pltpu.SMEM),
            pl.BlockSpec(memory_space=pltpu.SMEM),
        ],
        out_shape=[
            jax.ShapeDtypeStruct((1, 1), jnp.float32),
            jax.ShapeDtypeStruct((1, 1), jnp.float32),
        ],
    )


def kernel(outputs, target):
    ts, tc = _make_tc()(outputs, target)
    p = _make_sc()(outputs, target)
    num = ts[0, 0] + jnp.sum(p[:, 0, :])
    den = tc[0, 0] + jnp.sum(p[:, 1, :])
    return num / den
